# baseline (speedup 1.0000x reference)
import functools

import jax
import jax.numpy as jnp
from jax import lax
from jax.experimental import pallas as pl
from jax.experimental.pallas import tpu as pltpu
from jax.experimental.pallas import tpu_sc as plsc

N = 10000
E = 320000
D = 128
DE = 16
G = 64

NUM_TILES = 32
CHUNK = 64
NCHUNKS = E // CHUNK
FULL_ROUNDS = NCHUNKS // NUM_TILES
EXTRA = NCHUNKS - FULL_ROUNDS * NUM_TILES
N_PAD = 10240
ROWS_PER_TILE = N_PAD // 16


def _edge_linear(edge_attr, w, b):
    BE = 4000

    def body(ea_ref, w_ref, b_ref, o_ref):
        o_ref[...] = (
            jnp.dot(ea_ref[...], w_ref[...],
                    preferred_element_type=jnp.float32) + b_ref[...])

    return pl.pallas_call(
        body,
        grid=(E // BE,),
        in_specs=[
            pl.BlockSpec((BE, DE), lambda i: (i, 0)),
            pl.BlockSpec((DE, D), lambda i: (0, 0)),
            pl.BlockSpec((1, D), lambda i: (0, 0)),
        ],
        out_specs=pl.BlockSpec((BE, D), lambda i: (i, 0)),
        out_shape=jax.ShapeDtypeStruct((E, D), jnp.float32),
    )(edge_attr, w, b.reshape(1, D))


def _make_sc_gather_scatter():
    mesh = plsc.VectorSubcoreMesh(core_axis_name="c", subcore_axis_name="s")
    NR = FULL_ROUNDS

    @functools.partial(
        pl.kernel,
        mesh=mesh,
        out_type=jax.ShapeDtypeStruct((2 * N_PAD, D), jnp.float32),
        scratch_types=[
            pltpu.VMEM((3, CHUNK, D), jnp.float32),
            pltpu.VMEM((2, CHUNK, D), jnp.float32),
            pltpu.VMEM((6, CHUNK), jnp.int32),
            pltpu.VMEM((6, CHUNK), jnp.int32),
            pltpu.VMEM_SHARED((N_PAD, D), jnp.float32),
            pltpu.SemaphoreType.DMA,
            pltpu.SemaphoreType.DMA,
            pltpu.SemaphoreType.DMA,
            pltpu.SemaphoreType.DMA,
            pltpu.SemaphoreType.DMA,
            pltpu.SemaphoreType.DMA,
            pltpu.SemaphoreType.DMA,
            pltpu.SemaphoreType.DMA,
            pltpu.SemaphoreType.DMA,
            pltpu.SemaphoreType.DMA,
            pltpu.SemaphoreType.DMA,
            pltpu.SemaphoreType.DMA,
            pltpu.SemaphoreType.DMA,
            pltpu.SemaphoreType.DMA,
        ],
    )
    def k(x_hbm, e_hbm, src_hbm, dst_hbm, out_hbm,
          xs, eb, srcv, dstv, agg_sh,
          g0, g1, g2, e0, e1, s0, s1, s2, i0, i1, i2, i3, i4, i5):
        gsem = (g0, g1, g2)
        esem = (e0, e1)
        ssem = (s0, s1, s2)
        isem = (i0, i1, i2, i3, i4, i5)
        c = lax.axis_index("c")
        s = lax.axis_index("s")
        wid = c * 16 + s
        start = wid * NR + jnp.minimum(wid, EXTRA)

        def zrow(i, _):
            for j in range(D // 16):
                xs[0, i, pl.ds(j * 16, 16)] = jnp.zeros((16,), jnp.float32)
            return 0
        lax.fori_loop(0, CHUNK, zrow, 0)
        row0 = s * ROWS_PER_TILE
        for kk in range(ROWS_PER_TILE // CHUNK):
            pltpu.sync_copy(xs.at[0],
                            agg_sh.at[pl.ds(row0 + kk * CHUNK, CHUNK)])
        plsc.subcore_barrier()

        def issue_idx(r, p6):
            off = (start + r) * CHUNK
            pltpu.async_copy(src_hbm.at[pl.ds(off, CHUNK)], srcv.at[p6],
                             isem[p6])
            pltpu.async_copy(dst_hbm.at[pl.ds(off, CHUNK)], dstv.at[p6],
                             isem[p6])

        def wait_idx(p6):
            for _ in range(2):
                pltpu.make_async_copy(
                    src_hbm.at[pl.ds(0, CHUNK)], srcv.at[p6], isem[p6]).wait()

        def issue_eload(r, p2):
            pltpu.async_copy(
                e_hbm.at[pl.ds((start + r) * CHUNK, CHUNK)], eb.at[p2],
                esem[p2])

        def issue_gather(p6, p3):
            pltpu.async_copy(x_hbm.at[srcv.at[p6]], xs.at[p3], gsem[p3])

        def wait_big(p3_or_buf, sem):
            pltpu.make_async_copy(
                e_hbm.at[pl.ds(0, CHUNK)], p3_or_buf, sem).wait()

        def compute(p3, p2):
            def rr(i, _):
                for u in range(4):
                    ri = 4 * i + u
                    for j in range(D // 16):
                        sl = pl.ds(j * 16, 16)
                        xs[p3, ri, sl] = jnp.maximum(
                            xs[p3, ri, sl] + eb[p2, ri, sl], 0.0)
                return 0
            lax.fori_loop(0, CHUNK // 4, rr, 0)

        issue_idx(0, 0)
        issue_idx(1, 1)
        issue_idx(2, 2)
        issue_eload(0, 0)
        issue_eload(1, 1)
        wait_idx(0)
        issue_gather(0, 0)

        def do_round(r, u, kk):
            p3, q3 = u % 3, (u + 1) % 3
            p2 = u % 2
            @pl.when(r + 1 < NR)
            def _():
                @pl.when(r >= 2)
                def _():
                    wait_big(xs.at[q3], ssem[q3])
                wait_idx((u + 1) % 6)
                issue_gather((u + 1) % 6, q3)

            wait_big(xs.at[p3], gsem[p3])
            wait_big(eb.at[p2], esem[p2])
            compute(p3, p2)
            pltpu.async_copy(xs.at[p3], agg_sh.at[dstv.at[u % 6]], ssem[p3],
                             add=True)

            @pl.when(r + 2 < NR)
            def _():
                issue_eload(r + 2, p2)

            @pl.when(r + 3 < NR)
            def _():
                issue_idx(r + 3, (u + 3) % 6)

        def six(kk, _):
            base = 6 * kk
            for u in range(6):
                do_round(base + u, u, kk)
            return 0
        lax.fori_loop(0, NR // 6, six, 0)

        wait_big(xs.at[(NR - 3) % 3], ssem[(NR - 3) % 3])
        wait_big(xs.at[(NR - 2) % 3], ssem[(NR - 2) % 3])
        wait_big(xs.at[(NR - 1) % 3], ssem[(NR - 1) % 3])

        @pl.when(wid < EXTRA)
        def _():
            off = (start + NR) * CHUNK
            pltpu.sync_copy(src_hbm.at[pl.ds(off, CHUNK)], srcv.at[0])
            pltpu.sync_copy(dst_hbm.at[pl.ds(off, CHUNK)], dstv.at[0])
            pltpu.sync_copy(e_hbm.at[pl.ds(off, CHUNK)], eb.at[0])
            pltpu.async_copy(x_hbm.at[srcv.at[0]], xs.at[0], gsem[0]).wait()
            compute(0, 0)
            pltpu.sync_copy(xs.at[0], agg_sh.at[dstv.at[0]], add=True)

        plsc.subcore_barrier()
        pltpu.sync_copy(
            agg_sh.at[pl.ds(row0, ROWS_PER_TILE)],
            out_hbm.at[pl.ds(c * N_PAD + row0, ROWS_PER_TILE)])

    return k


_sc_gather_scatter = _make_sc_gather_scatter()


BN = 2048
NB = N_PAD // BN


def _node_mlp(x, agg, w1, b1, w2, b2):
    def body(x_ref, a0_ref, a1_ref, w1_ref, b1_ref, w2_ref, b2_ref, h_ref):
        t = x_ref[...] + a0_ref[...] + a1_ref[...]
        u = jnp.maximum(
            jnp.dot(t, w1_ref[...],
                    preferred_element_type=jnp.float32) + b1_ref[...], 0.0)
        h_ref[...] = jnp.maximum(
            jnp.dot(u, w2_ref[...],
                    preferred_element_type=jnp.float32) + b2_ref[...], 0.0)

    return pl.pallas_call(
        body,
        grid=(NB,),
        in_specs=[
            pl.BlockSpec((BN, D), lambda i: (i, 0)),
            pl.BlockSpec((BN, D), lambda i: (i, 0)),
            pl.BlockSpec((BN, D), lambda i: (i + NB, 0)),
            pl.BlockSpec((D, D), lambda i: (0, 0)),
            pl.BlockSpec((1, D), lambda i: (0, 0)),
            pl.BlockSpec((D, D), lambda i: (0, 0)),
            pl.BlockSpec((1, D), lambda i: (0, 0)),
        ],
        out_specs=pl.BlockSpec((BN, D), lambda i: (i, 0)),
        out_shape=jax.ShapeDtypeStruct((N_PAD, D), jnp.float32),
    )(x, agg, agg, w1, b1.reshape(1, D), w2, b2.reshape(1, D))


def _mlp_pool_head(h1, agg, batch3, w1, b1, w2, b2,
                   fc1_w, fc1_b, fc2_w, fc2_b):
    def body(h_ref, a0_ref, a1_ref, b_ref, w1_ref, b1_ref, w2_ref, b2_ref,
             f1w_ref, f1b_ref, f2w_ref, f2b_ref, o_ref, acc_ref, cnt_ref):
        i = pl.program_id(0)

        @pl.when(i == 0)
        def _():
            acc_ref[...] = jnp.zeros((G, D), jnp.float32)
            cnt_ref[...] = jnp.zeros((G, D), jnp.float32)

        t = h_ref[...] + a0_ref[...] + a1_ref[...]
        u = jnp.maximum(
            jnp.dot(t, w1_ref[...],
                    preferred_element_type=jnp.float32) + b1_ref[...], 0.0)
        h2 = jnp.maximum(
            jnp.dot(u, w2_ref[...],
                    preferred_element_type=jnp.float32) + b2_ref[...], 0.0)

        gid = lax.broadcasted_iota(jnp.int32, (BN, G), 1)
        bvals = b_ref[0, 0, :]
        onehot = (bvals[:, None] == gid).astype(jnp.float32)
        acc_ref[...] += lax.dot_general(
            onehot, h2, (((0,), (0,)), ((), ())),
            preferred_element_type=jnp.float32)
        cnt_ref[...] += lax.dot_general(
            onehot, jnp.ones((BN, D), jnp.float32), (((0,), (0,)), ((), ())),
            preferred_element_type=jnp.float32)

        pooled = acc_ref[...] / jnp.maximum(cnt_ref[...], 1.0)
        uo = jnp.maximum(
            jnp.dot(pooled, f1w_ref[...],
                    preferred_element_type=jnp.float32) + f1b_ref[...], 0.0)
        o_ref[...] = jnp.dot(
            uo, f2w_ref[...], preferred_element_type=jnp.float32) + f2b_ref[...]

    return pl.pallas_call(
        body,
        grid=(NB,),
        in_specs=[
            pl.BlockSpec((BN, D), lambda i: (i, 0)),
            pl.BlockSpec((BN, D), lambda i: (i, 0)),
            pl.BlockSpec((BN, D), lambda i: (i + NB, 0)),
            pl.BlockSpec((1, 1, BN), lambda i: (i, 0, 0)),
            pl.BlockSpec((D, D), lambda i: (0, 0)),
            pl.BlockSpec((1, D), lambda i: (0, 0)),
            pl.BlockSpec((D, D), lambda i: (0, 0)),
            pl.BlockSpec((1, D), lambda i: (0, 0)),
            pl.BlockSpec((D, D), lambda i: (0, 0)),
            pl.BlockSpec((1, D), lambda i: (0, 0)),
            pl.BlockSpec((D, 1), lambda i: (0, 0)),
            pl.BlockSpec((1, 1), lambda i: (0, 0)),
        ],
        out_specs=pl.BlockSpec((G, 1), lambda i: (0, 0)),
        out_shape=jax.ShapeDtypeStruct((G, 1), jnp.float32),
        scratch_shapes=[
            pltpu.VMEM((G, D), jnp.float32),
            pltpu.VMEM((G, D), jnp.float32),
        ],
    )(h1, agg, agg, batch3, w1, b1.reshape(1, D), w2, b2.reshape(1, D),
      fc1_w, fc1_b.reshape(1, D), fc2_w, fc2_b.reshape(1, 1))


def kernel(x, edge_index, edge_attr, batch, le1_w, le1_b, c1_w1, c1_b1,
           c1_w2, c1_b2, le2_w, le2_b, c2_w1, c2_b1, c2_w2, c2_b2,
           fc1_w, fc1_b, fc2_w, fc2_b):
    x_pad = jnp.pad(x, ((0, N_PAD - N), (0, 0)))
    batch3 = jnp.pad(batch, (0, N_PAD - N),
                     constant_values=G).reshape(NB, 1, BN)
    src = edge_index[0]
    dst = edge_index[1]

    e1 = _edge_linear(edge_attr, le1_w, le1_b)
    agg = _sc_gather_scatter(x_pad, e1, src, dst)
    e2 = _edge_linear(edge_attr, le2_w, le2_b)
    h1 = _node_mlp(x_pad, agg, c1_w1, c1_b1, c1_w2, c1_b2)
    agg2 = _sc_gather_scatter(h1, e2, src, dst)
    o = _mlp_pool_head(h1, agg2, batch3, c2_w1, c2_b1, c2_w2, c2_b2,
                       fc1_w, fc1_b, fc2_w, fc2_b)
    return jnp.squeeze(o, axis=-1)


# speedup vs baseline: 1.0190x; 1.0190x over previous
import functools

import jax
import jax.numpy as jnp
from jax import lax
from jax.experimental import pallas as pl
from jax.experimental.pallas import tpu as pltpu
from jax.experimental.pallas import tpu_sc as plsc

N = 10000
E = 320000
D = 128
DE = 16
G = 64

NUM_TILES = 32
CHUNK = 64
NCHUNKS = E // CHUNK
FULL_ROUNDS = NCHUNKS // NUM_TILES
EXTRA = NCHUNKS - FULL_ROUNDS * NUM_TILES
N_PAD = 10240
ROWS_PER_TILE = N_PAD // 16


def _edge_linear(edge_attr, w1, b1, w2, b2):
    BE = 4000

    def body(ea_ref, w1_ref, b1_ref, w2_ref, b2_ref, o1_ref, o2_ref):
        ea = ea_ref[...]
        o1_ref[...] = (
            jnp.dot(ea, w1_ref[...],
                    preferred_element_type=jnp.float32) + b1_ref[...])
        o2_ref[...] = (
            jnp.dot(ea, w2_ref[...],
                    preferred_element_type=jnp.float32) + b2_ref[...])

    return pl.pallas_call(
        body,
        grid=(E // BE,),
        in_specs=[
            pl.BlockSpec((BE, DE), lambda i: (i, 0)),
            pl.BlockSpec((DE, D), lambda i: (0, 0)),
            pl.BlockSpec((1, D), lambda i: (0, 0)),
            pl.BlockSpec((DE, D), lambda i: (0, 0)),
            pl.BlockSpec((1, D), lambda i: (0, 0)),
        ],
        out_specs=[
            pl.BlockSpec((BE, D), lambda i: (i, 0)),
            pl.BlockSpec((BE, D), lambda i: (i, 0)),
        ],
        out_shape=[
            jax.ShapeDtypeStruct((E, D), jnp.float32),
            jax.ShapeDtypeStruct((E, D), jnp.float32),
        ],
    )(edge_attr, w1, b1.reshape(1, D), w2, b2.reshape(1, D))


def _make_sc_gather_scatter():
    mesh = plsc.VectorSubcoreMesh(core_axis_name="c", subcore_axis_name="s")
    NR = FULL_ROUNDS

    @functools.partial(
        pl.kernel,
        mesh=mesh,
        out_type=jax.ShapeDtypeStruct((2 * N_PAD, D), jnp.float32),
        scratch_types=[
            pltpu.VMEM((3, CHUNK, D), jnp.float32),
            pltpu.VMEM((2, CHUNK, D), jnp.float32),
            pltpu.VMEM((6, CHUNK), jnp.int32),
            pltpu.VMEM((6, CHUNK), jnp.int32),
            pltpu.VMEM_SHARED((N_PAD, D), jnp.float32),
            pltpu.SemaphoreType.DMA,
            pltpu.SemaphoreType.DMA,
            pltpu.SemaphoreType.DMA,
            pltpu.SemaphoreType.DMA,
            pltpu.SemaphoreType.DMA,
            pltpu.SemaphoreType.DMA,
            pltpu.SemaphoreType.DMA,
            pltpu.SemaphoreType.DMA,
            pltpu.SemaphoreType.DMA,
            pltpu.SemaphoreType.DMA,
            pltpu.SemaphoreType.DMA,
            pltpu.SemaphoreType.DMA,
            pltpu.SemaphoreType.DMA,
            pltpu.SemaphoreType.DMA,
        ],
    )
    def k(x_hbm, e_hbm, src_hbm, dst_hbm, out_hbm,
          xs, eb, srcv, dstv, agg_sh,
          g0, g1, g2, e0, e1, s0, s1, s2, i0, i1, i2, i3, i4, i5):
        gsem = (g0, g1, g2)
        esem = (e0, e1)
        ssem = (s0, s1, s2)
        isem = (i0, i1, i2, i3, i4, i5)
        c = lax.axis_index("c")
        s = lax.axis_index("s")
        wid = c * 16 + s
        start = wid * NR + jnp.minimum(wid, EXTRA)

        def zrow(i, _):
            for j in range(D // 16):
                xs[0, i, pl.ds(j * 16, 16)] = jnp.zeros((16,), jnp.float32)
            return 0
        lax.fori_loop(0, CHUNK, zrow, 0)
        row0 = s * ROWS_PER_TILE
        for kk in range(ROWS_PER_TILE // CHUNK):
            pltpu.sync_copy(xs.at[0],
                            agg_sh.at[pl.ds(row0 + kk * CHUNK, CHUNK)])
        plsc.subcore_barrier()

        def issue_idx(r, p6):
            off = (start + r) * CHUNK
            pltpu.async_copy(src_hbm.at[pl.ds(off, CHUNK)], srcv.at[p6],
                             isem[p6])
            pltpu.async_copy(dst_hbm.at[pl.ds(off, CHUNK)], dstv.at[p6],
                             isem[p6])

        def wait_idx(p6):
            for _ in range(2):
                pltpu.make_async_copy(
                    src_hbm.at[pl.ds(0, CHUNK)], srcv.at[p6], isem[p6]).wait()

        def issue_eload(r, p2):
            pltpu.async_copy(
                e_hbm.at[pl.ds((start + r) * CHUNK, CHUNK)], eb.at[p2],
                esem[p2])

        def issue_gather(p6, p3):
            pltpu.async_copy(x_hbm.at[srcv.at[p6]], xs.at[p3], gsem[p3])

        def wait_big(p3_or_buf, sem):
            pltpu.make_async_copy(
                e_hbm.at[pl.ds(0, CHUNK)], p3_or_buf, sem).wait()

        def compute(p3, p2):
            def rr(i, _):
                for u in range(4):
                    ri = 4 * i + u
                    for j in range(D // 16):
                        sl = pl.ds(j * 16, 16)
                        xs[p3, ri, sl] = jnp.maximum(
                            xs[p3, ri, sl] + eb[p2, ri, sl], 0.0)
                return 0
            lax.fori_loop(0, CHUNK // 4, rr, 0)

        issue_idx(0, 0)
        issue_idx(1, 1)
        issue_idx(2, 2)
        issue_eload(0, 0)
        issue_eload(1, 1)
        wait_idx(0)
        issue_gather(0, 0)

        def do_round(r, u, kk):
            p3, q3 = u % 3, (u + 1) % 3
            p2 = u % 2
            @pl.when(r + 1 < NR)
            def _():
                @pl.when(r >= 2)
                def _():
                    wait_big(xs.at[q3], ssem[q3])
                wait_idx((u + 1) % 6)
                issue_gather((u + 1) % 6, q3)

            wait_big(xs.at[p3], gsem[p3])
            wait_big(eb.at[p2], esem[p2])
            compute(p3, p2)
            pltpu.async_copy(xs.at[p3], agg_sh.at[dstv.at[u % 6]], ssem[p3],
                             add=True)

            @pl.when(r + 2 < NR)
            def _():
                issue_eload(r + 2, p2)

            @pl.when(r + 3 < NR)
            def _():
                issue_idx(r + 3, (u + 3) % 6)

        def six(kk, _):
            base = 6 * kk
            for u in range(6):
                do_round(base + u, u, kk)
            return 0
        lax.fori_loop(0, NR // 6, six, 0)

        wait_big(xs.at[(NR - 3) % 3], ssem[(NR - 3) % 3])
        wait_big(xs.at[(NR - 2) % 3], ssem[(NR - 2) % 3])
        wait_big(xs.at[(NR - 1) % 3], ssem[(NR - 1) % 3])

        @pl.when(wid < EXTRA)
        def _():
            off = (start + NR) * CHUNK
            pltpu.sync_copy(src_hbm.at[pl.ds(off, CHUNK)], srcv.at[0])
            pltpu.sync_copy(dst_hbm.at[pl.ds(off, CHUNK)], dstv.at[0])
            pltpu.sync_copy(e_hbm.at[pl.ds(off, CHUNK)], eb.at[0])
            pltpu.async_copy(x_hbm.at[srcv.at[0]], xs.at[0], gsem[0]).wait()
            compute(0, 0)
            pltpu.sync_copy(xs.at[0], agg_sh.at[dstv.at[0]], add=True)

        plsc.subcore_barrier()
        pltpu.sync_copy(
            agg_sh.at[pl.ds(row0, ROWS_PER_TILE)],
            out_hbm.at[pl.ds(c * N_PAD + row0, ROWS_PER_TILE)])

    return k


_sc_gather_scatter = _make_sc_gather_scatter()


BN = 2048
NB = N_PAD // BN


def _node_mlp(x, agg, w1, b1, w2, b2):
    def body(x_ref, a0_ref, a1_ref, w1_ref, b1_ref, w2_ref, b2_ref, h_ref):
        t = x_ref[...] + a0_ref[...] + a1_ref[...]
        u = jnp.maximum(
            jnp.dot(t, w1_ref[...],
                    preferred_element_type=jnp.float32) + b1_ref[...], 0.0)
        h_ref[...] = jnp.maximum(
            jnp.dot(u, w2_ref[...],
                    preferred_element_type=jnp.float32) + b2_ref[...], 0.0)

    return pl.pallas_call(
        body,
        grid=(NB,),
        in_specs=[
            pl.BlockSpec((BN, D), lambda i: (i, 0)),
            pl.BlockSpec((BN, D), lambda i: (i, 0)),
            pl.BlockSpec((BN, D), lambda i: (i + NB, 0)),
            pl.BlockSpec((D, D), lambda i: (0, 0)),
            pl.BlockSpec((1, D), lambda i: (0, 0)),
            pl.BlockSpec((D, D), lambda i: (0, 0)),
            pl.BlockSpec((1, D), lambda i: (0, 0)),
        ],
        out_specs=pl.BlockSpec((BN, D), lambda i: (i, 0)),
        out_shape=jax.ShapeDtypeStruct((N_PAD, D), jnp.float32),
    )(x, agg, agg, w1, b1.reshape(1, D), w2, b2.reshape(1, D))


def _mlp_pool_head(h1, agg, batch3, w1, b1, w2, b2,
                   fc1_w, fc1_b, fc2_w, fc2_b):
    def body(h_ref, a0_ref, a1_ref, b_ref, w1_ref, b1_ref, w2_ref, b2_ref,
             f1w_ref, f1b_ref, f2w_ref, f2b_ref, o_ref, acc_ref, cnt_ref):
        i = pl.program_id(0)

        @pl.when(i == 0)
        def _():
            acc_ref[...] = jnp.zeros((G, D), jnp.float32)
            cnt_ref[...] = jnp.zeros((G, D), jnp.float32)

        t = h_ref[...] + a0_ref[...] + a1_ref[...]
        u = jnp.maximum(
            jnp.dot(t, w1_ref[...],
                    preferred_element_type=jnp.float32) + b1_ref[...], 0.0)
        h2 = jnp.maximum(
            jnp.dot(u, w2_ref[...],
                    preferred_element_type=jnp.float32) + b2_ref[...], 0.0)

        gid = lax.broadcasted_iota(jnp.int32, (BN, G), 1)
        bvals = b_ref[0, 0, :]
        onehot = (bvals[:, None] == gid).astype(jnp.float32)
        acc_ref[...] += lax.dot_general(
            onehot, h2, (((0,), (0,)), ((), ())),
            preferred_element_type=jnp.float32)
        cnt_ref[...] += lax.dot_general(
            onehot, jnp.ones((BN, D), jnp.float32), (((0,), (0,)), ((), ())),
            preferred_element_type=jnp.float32)

        pooled = acc_ref[...] / jnp.maximum(cnt_ref[...], 1.0)
        uo = jnp.maximum(
            jnp.dot(pooled, f1w_ref[...],
                    preferred_element_type=jnp.float32) + f1b_ref[...], 0.0)
        o_ref[...] = jnp.dot(
            uo, f2w_ref[...], preferred_element_type=jnp.float32) + f2b_ref[...]

    return pl.pallas_call(
        body,
        grid=(NB,),
        in_specs=[
            pl.BlockSpec((BN, D), lambda i: (i, 0)),
            pl.BlockSpec((BN, D), lambda i: (i, 0)),
            pl.BlockSpec((BN, D), lambda i: (i + NB, 0)),
            pl.BlockSpec((1, 1, BN), lambda i: (i, 0, 0)),
            pl.BlockSpec((D, D), lambda i: (0, 0)),
            pl.BlockSpec((1, D), lambda i: (0, 0)),
            pl.BlockSpec((D, D), lambda i: (0, 0)),
            pl.BlockSpec((1, D), lambda i: (0, 0)),
            pl.BlockSpec((D, D), lambda i: (0, 0)),
            pl.BlockSpec((1, D), lambda i: (0, 0)),
            pl.BlockSpec((D, 1), lambda i: (0, 0)),
            pl.BlockSpec((1, 1), lambda i: (0, 0)),
        ],
        out_specs=pl.BlockSpec((G, 1), lambda i: (0, 0)),
        out_shape=jax.ShapeDtypeStruct((G, 1), jnp.float32),
        scratch_shapes=[
            pltpu.VMEM((G, D), jnp.float32),
            pltpu.VMEM((G, D), jnp.float32),
        ],
    )(h1, agg, agg, batch3, w1, b1.reshape(1, D), w2, b2.reshape(1, D),
      fc1_w, fc1_b.reshape(1, D), fc2_w, fc2_b.reshape(1, 1))


def kernel(x, edge_index, edge_attr, batch, le1_w, le1_b, c1_w1, c1_b1,
           c1_w2, c1_b2, le2_w, le2_b, c2_w1, c2_b1, c2_w2, c2_b2,
           fc1_w, fc1_b, fc2_w, fc2_b):
    x_pad = jnp.pad(x, ((0, N_PAD - N), (0, 0)))
    batch3 = jnp.pad(batch, (0, N_PAD - N),
                     constant_values=G).reshape(NB, 1, BN)
    src = edge_index[0]
    dst = edge_index[1]

    e1, e2 = _edge_linear(edge_attr, le1_w, le1_b, le2_w, le2_b)
    agg = _sc_gather_scatter(x_pad, e1, src, dst)
    h1 = _node_mlp(x_pad, agg, c1_w1, c1_b1, c1_w2, c1_b2)
    agg2 = _sc_gather_scatter(h1, e2, src, dst)
    o = _mlp_pool_head(h1, agg2, batch3, c2_w1, c2_b1, c2_w2, c2_b2,
                       fc1_w, fc1_b, fc2_w, fc2_b)
    return jnp.squeeze(o, axis=-1)
